# Initial kernel scaffold; baseline (speedup 1.0000x reference)
import functools

import jax
import jax.numpy as jnp
from jax import lax
from jax.experimental import pallas as pl
from jax.experimental.pallas import tpu as pltpu
from jax.experimental.pallas import tpu_sc as plsc

N_NODES = 10000
N_EDGES = 320000
D = 128

NC = 2
NS = 16
L = 16
NW = NC * NS

EDGES_PER_W = N_EDGES // NW
CHUNK = 80
NCHUNKS = EDGES_PER_W // CHUNK
ROWS_PER_TILE = N_NODES // NS


def _mm_body(x_ref, w_ref, o_ref):
    o_ref[...] = jnp.dot(x_ref[...], w_ref[...],
                         preferred_element_type=jnp.float32)


def _matmul(x, w):
    blk = 1000
    return pl.pallas_call(
        _mm_body,
        grid=(N_NODES // blk,),
        in_specs=[
            pl.BlockSpec((blk, D), lambda i: (i, 0)),
            pl.BlockSpec((D, D), lambda i: (0, 0)),
        ],
        out_specs=pl.BlockSpec((blk, D), lambda i: (i, 0)),
        out_shape=jax.ShapeDtypeStruct((N_NODES, D), jnp.float32),
    )(x, w)


def _edge_body(h_hbm, col_hbm, dst_hbm, val_hbm, out_hbm,
               col_v, dst_v, val_v, rows_v, acc, sem):
    c = lax.axis_index("c")
    s = lax.axis_index("s")
    wid = s * NC + c
    base = wid * EDGES_PER_W

    def _zero(i, _):
        rows_v[pl.ds(i * L, L)] = jnp.zeros((L,), jnp.float32)
        return 0
    lax.fori_loop(0, CHUNK * D // L, _zero, 0)
    rows2d = rows_v.reshape(CHUNK, D)
    row0 = s * ROWS_PER_TILE
    done = 0
    for step in (CHUNK,) * (ROWS_PER_TILE // CHUNK) + (ROWS_PER_TILE % CHUNK,):
        if step:
            pltpu.sync_copy(rows2d.at[pl.ds(0, step)],
                            acc.at[pl.ds(row0 + done, step)])
            done += step
    plsc.subcore_barrier()

    def _chunk(j, _):
        off = base + j * CHUNK
        pltpu.sync_copy(col_hbm.at[pl.ds(off, CHUNK)], col_v)
        pltpu.sync_copy(dst_hbm.at[pl.ds(off, CHUNK)], dst_v)
        pltpu.sync_copy(val_hbm.at[pl.ds(off, CHUNK)], val_v)
        pltpu.async_copy(h_hbm.at[col_v], rows2d, sem).wait()

        def _scale(k, _):
            v = val_v[k]
            for d_ in range(D // L):
                sl = pl.ds(d_ * L, L)
                rows2d[k, sl] = rows2d[k, sl] * v
            return 0
        lax.fori_loop(0, CHUNK, _scale, 0)

        pltpu.sync_copy(rows2d, acc.at[dst_v], add=True)
        return 0

    lax.fori_loop(0, NCHUNKS, _chunk, 0)
    plsc.subcore_barrier()

    pltpu.sync_copy(acc.at[pl.ds(row0, ROWS_PER_TILE)],
                    out_hbm.at[c, pl.ds(row0, ROWS_PER_TILE)])


_edge_pass = functools.partial(
    pl.kernel,
    out_type=jax.ShapeDtypeStruct((NC, N_NODES, D), jnp.float32),
    mesh=plsc.VectorSubcoreMesh(core_axis_name="c", subcore_axis_name="s"),
    scratch_types=[
        pltpu.VMEM((CHUNK,), jnp.int32),
        pltpu.VMEM((CHUNK,), jnp.int32),
        pltpu.VMEM((CHUNK,), jnp.float32),
        pltpu.VMEM((CHUNK * D,), jnp.float32),
        pltpu.VMEM_SHARED((N_NODES, D), jnp.float32),
        pltpu.SemaphoreType.DMA,
    ],
)(_edge_body)


def _comb_body(p_ref, o_ref):
    o_ref[...] = jnp.maximum(p_ref[0] + p_ref[1], 0.0)


def _combine(partials):
    blk = 1000
    return pl.pallas_call(
        _comb_body,
        grid=(N_NODES // blk,),
        in_specs=[pl.BlockSpec((NC, blk, D), lambda i: (0, i, 0))],
        out_specs=pl.BlockSpec((blk, D), lambda i: (i, 0)),
        out_shape=jax.ShapeDtypeStruct((N_NODES, D), jnp.float32),
    )(partials)


def kernel(inputs, edge_index, adj_vals, W):
    h = _matmul(inputs, W)
    dst = edge_index[0].astype(jnp.int32)
    col = edge_index[1].astype(jnp.int32)
    partials = _edge_pass(h, col, dst, adj_vals)
    return _combine(partials)


# speedup vs baseline: 4.4398x; 4.4398x over previous
import functools

import jax
import jax.numpy as jnp
from jax import lax
from jax.experimental import pallas as pl
from jax.experimental.pallas import tpu as pltpu
from jax.experimental.pallas import tpu_sc as plsc

N_NODES = 10000
N_EDGES = 320000
D = 128

NC = 2
NS = 16
L = 16
NW = NC * NS

EDGES_PER_W = N_EDGES // NW
CHUNK = 80
NCHUNKS = EDGES_PER_W // CHUNK
WB_TILES = 10
WB_ROWS = N_NODES // WB_TILES


def _mm_body(x_ref, w_ref, o_ref):
    o_ref[...] = jnp.dot(x_ref[...], w_ref[...],
                         preferred_element_type=jnp.float32)


def _matmul(x, w):
    blk = 1000
    return pl.pallas_call(
        _mm_body,
        grid=(N_NODES // blk,),
        in_specs=[
            pl.BlockSpec((blk, D), lambda i: (i, 0)),
            pl.BlockSpec((D, D), lambda i: (0, 0)),
        ],
        out_specs=pl.BlockSpec((blk, D), lambda i: (i, 0)),
        out_shape=jax.ShapeDtypeStruct((N_NODES, D), jnp.float32),
    )(x, w)


def _edge_body(h_hbm, col_hbm, dst_hbm, val_hbm, out_hbm,
               col_v, dst_v, val_v, rows_v, acc, sem):
    c = lax.axis_index("c")
    s = lax.axis_index("s")
    wid = s * NC + c
    base = wid * EDGES_PER_W

    rows2d = rows_v
    zero = jnp.zeros((L,), jnp.float32)

    def _zero(k, _):
        for d_ in range(D // L):
            rows2d[k, pl.ds(d_ * L, L)] = zero
        return 0
    lax.fori_loop(0, CHUNK, _zero, 0)
    row0 = s * WB_ROWS

    @pl.when(s < WB_TILES)
    def _init_acc():
        done = 0
        for step in (CHUNK,) * (WB_ROWS // CHUNK) + (WB_ROWS % CHUNK,):
            if step:
                pltpu.sync_copy(rows2d.at[pl.ds(0, step)],
                                acc.at[pl.ds(row0 + done, step)])
                done += step
    plsc.subcore_barrier()

    def _chunk(j, _):
        off = base + j * CHUNK
        pltpu.sync_copy(col_hbm.at[pl.ds(off, CHUNK)], col_v)
        pltpu.sync_copy(dst_hbm.at[pl.ds(off, CHUNK)], dst_v)
        pltpu.sync_copy(val_hbm.at[pl.ds(off, CHUNK)], val_v)
        pltpu.async_copy(h_hbm.at[col_v], rows2d, sem).wait()

        def _scale(k0, _):
            v16 = val_v[pl.ds(k0 * L, L)]
            for e in range(L):
                v = v16[e]
                k = k0 * L + e
                for d_ in range(D // L):
                    sl = pl.ds(d_ * L, L)
                    rows2d[k, sl] = rows2d[k, sl] * v
            return 0
        lax.fori_loop(0, CHUNK // L, _scale, 0)

        pltpu.sync_copy(rows2d, acc.at[dst_v], add=True)
        return 0

    lax.fori_loop(0, NCHUNKS, _chunk, 0)
    plsc.subcore_barrier()

    @pl.when(s < WB_TILES)
    def _writeback():
        pltpu.sync_copy(acc.at[pl.ds(row0, WB_ROWS)],
                        out_hbm.at[c, pl.ds(row0, WB_ROWS)])


_edge_pass = functools.partial(
    pl.kernel,
    out_type=jax.ShapeDtypeStruct((NC, N_NODES, D), jnp.float32),
    mesh=plsc.VectorSubcoreMesh(core_axis_name="c", subcore_axis_name="s"),
    scratch_types=[
        pltpu.VMEM((CHUNK,), jnp.int32),
        pltpu.VMEM((CHUNK,), jnp.int32),
        pltpu.VMEM((CHUNK,), jnp.float32),
        pltpu.VMEM((CHUNK, D), jnp.float32),
        pltpu.VMEM_SHARED((N_NODES, D), jnp.float32),
        pltpu.SemaphoreType.DMA,
    ],
)(_edge_body)


def _comb_body(p_ref, o_ref):
    o_ref[...] = jnp.maximum(p_ref[0] + p_ref[1], 0.0)


def _combine(partials):
    blk = 1000
    return pl.pallas_call(
        _comb_body,
        grid=(N_NODES // blk,),
        in_specs=[pl.BlockSpec((NC, blk, D), lambda i: (0, i, 0))],
        out_specs=pl.BlockSpec((blk, D), lambda i: (i, 0)),
        out_shape=jax.ShapeDtypeStruct((N_NODES, D), jnp.float32),
    )(partials)


def kernel(inputs, edge_index, adj_vals, W):
    h = _matmul(inputs, W)
    dst = edge_index[0].astype(jnp.int32)
    col = edge_index[1].astype(jnp.int32)
    partials = _edge_pass(h, col, dst, adj_vals)
    return _combine(partials)
